# baseline (speedup 1.0000x reference)
import functools

import jax
import jax.numpy as jnp
from jax import lax
from jax.experimental import pallas as pl
from jax.experimental.pallas import tpu as pltpu
from jax.experimental.pallas import tpu_sc as plsc

F = 64
L = 16
TILE = 8
CHUNK = 32


def _mf_body(u_idx_hbm, i_idx_hbm, u3_hbm, i3_hbm, out_hbm,
             uidx_v, iidx_v, ubuf, ibuf, out_v, sem0, sem1):
    nc = 2
    wid = lax.axis_index("s") * nc + lax.axis_index("c")
    b_per_w = out_v.shape[0]
    n_chunks = b_per_w // CHUNK
    base = wid * b_per_w

    pltpu.sync_copy(u_idx_hbm.at[pl.ds(base, b_per_w)], uidx_v)
    pltpu.sync_copy(i_idx_hbm.at[pl.ds(base, b_per_w)], iidx_v)

    lanes = lax.broadcasted_iota(jnp.int32, (L,), 0)
    sems = (sem0, sem1)

    def issue(c, st):
        for h in range(CHUNK // L):
            uvec = uidx_v[pl.ds(c * CHUNK + h * L, L)]
            ivec = iidx_v[pl.ds(c * CHUNK + h * L, L)]
            ut = lax.shift_right_logical(uvec, 3)
            it = lax.shift_right_logical(ivec, 3)
            usub = uvec & (TILE - 1)
            isub = ivec & (TILE - 1)
            for j in range(L):
                pltpu.async_copy(
                    u3_hbm.at[pl.ds(ut[j], 1), pl.ds(usub[j], 1)],
                    ubuf.at[st, pl.ds(h * L + j, 1)], sems[st])
                pltpu.async_copy(
                    i3_hbm.at[pl.ds(it[j], 1), pl.ds(isub[j], 1)],
                    ibuf.at[st, pl.ds(h * L + j, 1)], sems[st])

    def drain(st):
        pltpu.make_async_copy(u3_hbm.at[pl.ds(0, CHUNK), pl.ds(0, 1)],
                              ubuf.at[st], sems[st]).wait()
        pltpu.make_async_copy(i3_hbm.at[pl.ds(0, CHUNK), pl.ds(0, 1)],
                              ibuf.at[st], sems[st]).wait()

    def compute(c, st):
        for h in range(CHUNK // L):
            res = jnp.zeros((L,), jnp.float32)
            for j in range(L):
                acc = jnp.zeros((L,), jnp.float32)
                for q in range(F // L):
                    uv = ubuf[st, h * L + j, 0, pl.ds(q * L, L)]
                    iv = ibuf[st, h * L + j, 0, pl.ds(q * L, L)]
                    acc = acc + uv * iv
                res = jnp.where(lanes == j, jnp.sum(acc), res)
            out_v[pl.ds(c * CHUNK + h * L, L)] = 1.0 / (1.0 + jnp.exp(-res))

    issue(0, 0)

    def body(k, _):
        c0 = k * 2
        drain(0)
        issue(c0 + 1, 1)
        compute(c0, 0)
        drain(1)

        @pl.when(c0 + 2 < n_chunks)
        def _():
            issue(c0 + 2, 0)

        compute(c0 + 1, 1)
        return 0

    lax.fori_loop(0, n_chunks // 2, body, 0)

    pltpu.sync_copy(out_v, out_hbm.at[pl.ds(base, b_per_w)])


def kernel(u_idx, i_idx, u_emb, i_emb):
    B = u_idx.shape[0]
    nw = 32
    b_per_w = B // nw
    u3 = u_emb.reshape(u_emb.shape[0] // TILE, TILE, F)
    i3 = i_emb.reshape(i_emb.shape[0] // TILE, TILE, F)
    mesh = plsc.VectorSubcoreMesh(core_axis_name="c", subcore_axis_name="s")

    mf = functools.partial(
        pl.kernel, mesh=mesh,
        out_type=jax.ShapeDtypeStruct((B,), jnp.float32),
        scratch_types=[
            pltpu.VMEM((b_per_w,), jnp.int32),
            pltpu.VMEM((b_per_w,), jnp.int32),
            pltpu.VMEM((2, CHUNK, 1, F), jnp.float32),
            pltpu.VMEM((2, CHUNK, 1, F), jnp.float32),
            pltpu.VMEM((b_per_w,), jnp.float32),
            pltpu.SemaphoreType.DMA,
            pltpu.SemaphoreType.DMA,
        ],
        compiler_params=pltpu.CompilerParams(needs_layout_passes=False),
    )(_mf_body)

    return mf(u_idx.astype(jnp.int32), i_idx.astype(jnp.int32), u3, i3)


# speedup vs baseline: 1.0042x; 1.0042x over previous
import functools

import jax
import jax.numpy as jnp
from jax import lax
from jax.experimental import pallas as pl
from jax.experimental.pallas import tpu as pltpu
from jax.experimental.pallas import tpu_sc as plsc

F = 64
L = 16
TILE = 8
CHUNK = 32


def _mf_body(u_idx_hbm, i_idx_hbm, u3_hbm, i3_hbm, out_hbm,
             uidx_v, iidx_v, ubuf, ibuf, out_v, sem0, sem1):
    nc = 2
    wid = lax.axis_index("s") * nc + lax.axis_index("c")
    b_per_w = out_v.shape[0]
    n_chunks = b_per_w // CHUNK
    base = wid * b_per_w

    pltpu.sync_copy(u_idx_hbm.at[pl.ds(base, b_per_w)], uidx_v)
    pltpu.sync_copy(i_idx_hbm.at[pl.ds(base, b_per_w)], iidx_v)

    lanes = lax.broadcasted_iota(jnp.int32, (L,), 0)
    sems = (sem0, sem1)

    def issue(c, st):
        for h in range(CHUNK // L):
            uvec = uidx_v[pl.ds(c * CHUNK + h * L, L)]
            ivec = iidx_v[pl.ds(c * CHUNK + h * L, L)]
            ut = lax.shift_right_logical(uvec, 3)
            it = lax.shift_right_logical(ivec, 3)
            usub = uvec & (TILE - 1)
            isub = ivec & (TILE - 1)
            for j in range(L):
                pltpu.async_copy(
                    u3_hbm.at[pl.ds(ut[j], 1), pl.ds(usub[j], 1)],
                    ubuf.at[st, pl.ds(h * L + j, 1)], sems[st])
                pltpu.async_copy(
                    i3_hbm.at[pl.ds(it[j], 1), pl.ds(isub[j], 1)],
                    ibuf.at[st, pl.ds(h * L + j, 1)], sems[st])

    def drain(st):
        pltpu.make_async_copy(u3_hbm.at[pl.ds(0, CHUNK), pl.ds(0, 1)],
                              ubuf.at[st], sems[st]).wait()
        pltpu.make_async_copy(i3_hbm.at[pl.ds(0, CHUNK), pl.ds(0, 1)],
                              ibuf.at[st], sems[st]).wait()

    def compute(c, st):
        for h in range(CHUNK // L):
            res = jnp.zeros((L,), jnp.float32)
            for j in range(L):
                acc = jnp.zeros((L,), jnp.float32)
                for q in range(F // L):
                    uv = ubuf[st, h * L + j, 0, pl.ds(q * L, L)]
                    iv = ibuf[st, h * L + j, 0, pl.ds(q * L, L)]
                    acc = acc + uv * iv
                res = jnp.where(lanes == j, jnp.sum(acc), res)
            out_v[pl.ds(c * CHUNK + h * L, L)] = 1.0 / (1.0 + jnp.exp(-res))

    issue(0, 0)

    def body(k, _):
        c0 = k * 2
        drain(0)
        issue(c0 + 1, 1)
        compute(c0, 0)

        @pl.when(c0 + 2 < n_chunks)
        def _():
            issue(c0 + 2, 0)

        drain(1)
        compute(c0 + 1, 1)
        return 0

    lax.fori_loop(0, n_chunks // 2, body, 0)

    pltpu.sync_copy(out_v, out_hbm.at[pl.ds(base, b_per_w)])


def kernel(u_idx, i_idx, u_emb, i_emb):
    B = u_idx.shape[0]
    nw = 32
    b_per_w = B // nw
    u3 = u_emb.reshape(u_emb.shape[0] // TILE, TILE, F)
    i3 = i_emb.reshape(i_emb.shape[0] // TILE, TILE, F)
    mesh = plsc.VectorSubcoreMesh(core_axis_name="c", subcore_axis_name="s")

    mf = functools.partial(
        pl.kernel, mesh=mesh,
        out_type=jax.ShapeDtypeStruct((B,), jnp.float32),
        scratch_types=[
            pltpu.VMEM((b_per_w,), jnp.int32),
            pltpu.VMEM((b_per_w,), jnp.int32),
            pltpu.VMEM((2, CHUNK, 1, F), jnp.float32),
            pltpu.VMEM((2, CHUNK, 1, F), jnp.float32),
            pltpu.VMEM((b_per_w,), jnp.float32),
            pltpu.SemaphoreType.DMA,
            pltpu.SemaphoreType.DMA,
        ],
        compiler_params=pltpu.CompilerParams(needs_layout_passes=False),
    )(_mf_body)

    return mf(u_idx.astype(jnp.int32), i_idx.astype(jnp.int32), u3, i3)
